# baseline (speedup 1.0000x reference)
import functools

import jax
import jax.numpy as jnp
from jax import lax
from jax.experimental import pallas as pl
from jax.experimental.pallas import tpu as pltpu
from jax.experimental.pallas import tpu_sc as plsc

K = 1024
D = 32
N = 2304
TBLK = 128
CHUNK = 8

_NC = 2
_NS = 16
_NW = _NC * _NS
_BPW = N // _NW


def _dist_argmin_kernel(z_ref, cbt_ref, idx_ref):
    iota_k = lax.broadcasted_iota(jnp.int32, (CHUNK, K), 1)
    for r in range(TBLK // CHUNK):
        zc = z_ref[pl.ds(r * CHUNK, CHUNK), :]

        def sq(d):
            diff = zc[:, d:d + 1] - cbt_ref[d:d + 1, :]
            return diff * diff

        def grp(g):
            b = 8 * g
            p04 = sq(b + 0) + sq(b + 4)
            p26 = sq(b + 2) + sq(b + 6)
            p15 = sq(b + 1) + sq(b + 5)
            p37 = sq(b + 3) + sq(b + 7)
            return (p04 + p26) + (p15 + p37)

        dist = ((grp(0) + grp(1)) + grp(2)) + grp(3)
        dmin = jnp.min(dist, axis=1, keepdims=True)
        cand = jnp.where(dist == dmin, iota_k, jnp.int32(K))
        idx = jnp.min(cand, axis=1, keepdims=True)
        idx_ref[pl.ds(r * CHUNK, CHUNK), :] = idx


def _argmin_indices(z_flat, cbt):
    idx2d = pl.pallas_call(
        _dist_argmin_kernel,
        grid=(N // TBLK,),
        in_specs=[
            pl.BlockSpec((TBLK, D), lambda i: (i, 0)),
            pl.BlockSpec((D, K), lambda i: (0, 0)),
        ],
        out_specs=pl.BlockSpec((TBLK, 1), lambda i: (i, 0)),
        out_shape=jax.ShapeDtypeStruct((N, 1), jnp.int32),
    )(z_flat, cbt)
    return idx2d.reshape(-1)


_DPAD = 128


@functools.cache
def _make_sc_gather():
    @functools.partial(
        pl.kernel,
        mesh=plsc.VectorSubcoreMesh(core_axis_name="c", subcore_axis_name="s"),
        out_type=jax.ShapeDtypeStruct((N, _DPAD), jnp.float32),
        scratch_types=[
            pltpu.VMEM((_BPW,), jnp.int32),
            pltpu.VMEM((_BPW, _DPAD), jnp.float32),
            pltpu.SemaphoreType.DMA,
        ],
    )
    def _sc_gather(cb_hbm, idx_hbm, out_hbm, idx_v, rows_v, sem):
        wid = lax.axis_index("s") * _NC + lax.axis_index("c")
        base = wid * _BPW
        pltpu.sync_copy(idx_hbm.at[pl.ds(base, _BPW)], idx_v)
        pltpu.async_copy(cb_hbm.at[idx_v], rows_v, sem).wait()
        pltpu.sync_copy(rows_v, out_hbm.at[pl.ds(base, _BPW)])

    return _sc_gather


def kernel(z_e, codebook):
    B, C, H, W = z_e.shape
    z_flat = jnp.transpose(z_e, (0, 2, 3, 1)).reshape(-1, C)
    indices = _argmin_indices(z_flat, codebook.T)
    cb_pad = jnp.pad(codebook, ((0, 0), (0, _DPAD - D)))
    zq_flat = _make_sc_gather()(cb_pad, indices)[:, :D]
    z_q = jnp.transpose(zq_flat.reshape(B, C, H, W), (0, 3, 1, 2))
    return (z_q, indices)


# speedup vs baseline: 1.8317x; 1.8317x over previous
import functools

import jax
import jax.numpy as jnp
from jax import lax
from jax.experimental import pallas as pl
from jax.experimental.pallas import tpu as pltpu
from jax.experimental.pallas import tpu_sc as plsc

K = 1024
D = 32
N = 2304
TBLK = 256
CHUNK = 8

_NC = 2
_NS = 16
_NW = _NC * _NS
_BPW = N // _NW


NPAIR = 32


def _dist_argmin_kernel(z_ref, cbt8_ref, idx_ref):
    iota_k = lax.broadcasted_iota(jnp.int32, (CHUNK, K), 1)
    for r in range(0, TBLK // CHUNK, NPAIR):
        zcs = [z_ref[pl.ds((r + j) * CHUNK, CHUNK), :] for j in range(NPAIR)]

        def sq(d):
            row = cbt8_ref[d]
            out = []
            for j in range(NPAIR):
                diff = zcs[j][:, d:d + 1] - row
                out.append(diff * diff)
            return out

        def vadd(a, b):
            return [x + y for x, y in zip(a, b)]

        def grp(g):
            b = 8 * g
            p04 = vadd(sq(b + 0), sq(b + 4))
            p26 = vadd(sq(b + 2), sq(b + 6))
            p15 = vadd(sq(b + 1), sq(b + 5))
            p37 = vadd(sq(b + 3), sq(b + 7))
            return vadd(vadd(p04, p26), vadd(p15, p37))

        dist = vadd(vadd(vadd(grp(0), grp(1)), grp(2)), grp(3))
        for j in range(NPAIR):
            dmin = jnp.min(dist[j], axis=1, keepdims=True)
            cand = jnp.where(dist[j] == dmin, iota_k, jnp.int32(K))
            idx = jnp.min(cand, axis=1, keepdims=True)
            idx_ref[pl.ds((r + j) * CHUNK, CHUNK), :] = idx


def _argmin_indices(z_flat, cbt8):
    idx2d = pl.pallas_call(
        _dist_argmin_kernel,
        grid=(N // TBLK,),
        in_specs=[
            pl.BlockSpec((TBLK, D), lambda i: (i, 0)),
            pl.BlockSpec((D, CHUNK, K), lambda i: (0, 0, 0)),
        ],
        out_specs=pl.BlockSpec((TBLK, 1), lambda i: (i, 0)),
        out_shape=jax.ShapeDtypeStruct((N, 1), jnp.int32),
    )(z_flat, cbt8)
    return idx2d.reshape(-1)


_DPAD = 128


@functools.cache
def _make_sc_gather():
    @functools.partial(
        pl.kernel,
        mesh=plsc.VectorSubcoreMesh(core_axis_name="c", subcore_axis_name="s"),
        out_type=jax.ShapeDtypeStruct((N, _DPAD), jnp.float32),
        scratch_types=[
            pltpu.VMEM((_BPW,), jnp.int32),
            pltpu.VMEM((_BPW, _DPAD), jnp.float32),
            pltpu.SemaphoreType.DMA,
        ],
    )
    def _sc_gather(cb_hbm, idx_hbm, out_hbm, idx_v, rows_v, sem):
        wid = lax.axis_index("s") * _NC + lax.axis_index("c")
        base = wid * _BPW
        pltpu.sync_copy(idx_hbm.at[pl.ds(base, _BPW)], idx_v)
        pltpu.async_copy(cb_hbm.at[idx_v], rows_v, sem).wait()
        pltpu.sync_copy(rows_v, out_hbm.at[pl.ds(base, _BPW)])

    return _sc_gather


def kernel(z_e, codebook):
    B, C, H, W = z_e.shape
    z_flat = jnp.transpose(z_e, (0, 2, 3, 1)).reshape(-1, C)
    cbt8 = jnp.broadcast_to(codebook.T[:, None, :], (D, CHUNK, K))
    indices = _argmin_indices(z_flat, cbt8)
    cb_pad = jnp.pad(codebook, ((0, 0), (0, _DPAD - D)))
    zq_flat = _make_sc_gather()(cb_pad, indices)[:, :D]
    z_q = jnp.transpose(zq_flat.reshape(B, C, H, W), (0, 3, 1, 2))
    return (z_q, indices)


# speedup vs baseline: 2.1473x; 1.1723x over previous
import jax
import jax.numpy as jnp
from jax import lax
from jax.experimental import pallas as pl

K = 1024
D = 32
N = 2304
TBLK = 256
CHUNK = 8
NPAIR = 32


def _vq_kernel(z_ref, cbt8_ref, cb_ref, idx_ref, zq_ref):
    iota_k = lax.broadcasted_iota(jnp.int32, (CHUNK, K), 1)
    for r in range(0, TBLK // CHUNK, NPAIR):
        zcs = [z_ref[pl.ds((r + j) * CHUNK, CHUNK), :] for j in range(NPAIR)]

        def sq(d):
            row = cbt8_ref[d]
            out = []
            for j in range(NPAIR):
                diff = zcs[j][:, d:d + 1] - row
                out.append(diff * diff)
            return out

        def vadd(a, b):
            return [x + y for x, y in zip(a, b)]

        def grp(g):
            b = 8 * g
            p04 = vadd(sq(b + 0), sq(b + 4))
            p26 = vadd(sq(b + 2), sq(b + 6))
            p15 = vadd(sq(b + 1), sq(b + 5))
            p37 = vadd(sq(b + 3), sq(b + 7))
            return vadd(vadd(p04, p26), vadd(p15, p37))

        dist = vadd(vadd(vadd(grp(0), grp(1)), grp(2)), grp(3))
        for j in range(NPAIR):
            dmin = jnp.min(dist[j], axis=1, keepdims=True)
            cand = jnp.where(dist[j] == dmin, iota_k, jnp.int32(K))
            idx = jnp.min(cand, axis=1, keepdims=True)
            idx_ref[pl.ds((r + j) * CHUNK, CHUNK), :] = idx
            onehot = (iota_k == idx).astype(jnp.float32)
            zq = jnp.dot(onehot, cb_ref[...],
                         preferred_element_type=jnp.float32)
            zq_ref[pl.ds((r + j) * CHUNK, CHUNK), :] = zq


def kernel(z_e, codebook):
    B, C, H, W = z_e.shape
    z_flat = jnp.transpose(z_e, (0, 2, 3, 1)).reshape(-1, C)
    cbt8 = jnp.broadcast_to(codebook.T[:, None, :], (D, CHUNK, K))
    idx2d, zq_flat = pl.pallas_call(
        _vq_kernel,
        grid=(N // TBLK,),
        in_specs=[
            pl.BlockSpec((TBLK, D), lambda i: (i, 0)),
            pl.BlockSpec((D, CHUNK, K), lambda i: (0, 0, 0)),
            pl.BlockSpec((K, D), lambda i: (0, 0)),
        ],
        out_specs=[
            pl.BlockSpec((TBLK, 1), lambda i: (i, 0)),
            pl.BlockSpec((TBLK, D), lambda i: (i, 0)),
        ],
        out_shape=[
            jax.ShapeDtypeStruct((N, 1), jnp.int32),
            jax.ShapeDtypeStruct((N, D), jnp.float32),
        ],
    )(z_flat, cbt8, codebook)
    indices = idx2d.reshape(-1)
    z_q = jnp.transpose(zq_flat.reshape(B, C, H, W), (0, 3, 1, 2))
    return (z_q, indices)
